# baseline (speedup 1.0000x reference)
import functools

import jax
import jax.numpy as jnp
from jax import lax
from jax.experimental import pallas as pl
from jax.experimental.pallas import tpu as pltpu
from jax.experimental.pallas import tpu_sc as plsc

N = 10000
E = 320000
D = 128
OUT = 40
EPS = 1e-5

NC = 2
NS = 16
E_PER_TILE = E // (NC * NS)
CHUNK = 128
NCHUNKS = E_PER_TILE // CHUNK
TAIL = E_PER_TILE - NCHUNKS * CHUNK
ZROWS = 40
WB_ROWS = 1000

_mesh = plsc.VectorSubcoreMesh(core_axis_name="c", subcore_axis_name="s")

MM_KW = dict(preferred_element_type=jnp.float32,
             precision=jax.lax.Precision.HIGHEST)


DCHUNK = 128
DNCHUNKS = E_PER_TILE // DCHUNK
DTAIL = E_PER_TILE - DNCHUNKS * DCHUNK
DNB = 2


@functools.partial(
    pl.kernel, mesh=_mesh,
    out_type=jax.ShapeDtypeStruct((NC * N,), jnp.float32),
    scratch_types=[
        pltpu.VMEM((DNB, DCHUNK), jnp.int32),
        pltpu.VMEM((DTAIL,), jnp.int32),
        pltpu.VMEM((DCHUNK,), jnp.float32),
        pltpu.VMEM((1024,), jnp.float32),
        pltpu.VMEM_SHARED((10240,), jnp.float32),
        pltpu.SemaphoreType.DMA,
        pltpu.SemaphoreType.DMA,
        pltpu.SemaphoreType.DMA,
        pltpu.SemaphoreType.DMA,
    ])
def _sc_degree(dst_hbm, out_hbm, dchunk, dtail, ones_v, zbuf, acc,
               d0, d1, s0, s1):
    c = lax.axis_index("c")
    s = lax.axis_index("s")
    t = c * NS + s
    base = t * E_PER_TILE
    dsem = (d0, d1)
    ssem = (s0, s1)
    z16 = jnp.zeros((16,), jnp.float32)
    o16 = jnp.ones((16,), jnp.float32)

    @pl.loop(0, DCHUNK, step=16)
    def _(i):
        ones_v[pl.ds(i, 16)] = o16

    @pl.loop(0, 1024, step=16)
    def _(i):
        zbuf[pl.ds(i, 16)] = z16

    @pl.when(s < 10)
    def _():
        pltpu.sync_copy(zbuf, acc.at[pl.ds(s * 1024, 1024)])

    plsc.subcore_barrier()

    pend = {}
    for k in range(DNCHUNKS):
        b = k % DNB
        if k >= DNB:
            pend[("s", b)].wait()
        pend[("d", b)] = pltpu.async_copy(
            dst_hbm.at[pl.ds(base + k * DCHUNK, DCHUNK)], dchunk.at[b],
            dsem[b])
        j = k - (DNB - 1)
        if j >= 0:
            bj = j % DNB
            pend[("d", bj)].wait()
            pend[("s", bj)] = pltpu.async_copy(
                ones_v, acc.at[dchunk.at[bj]], ssem[bj], add=True)
    for j in range(max(0, DNCHUNKS - DNB + 1), DNCHUNKS):
        bj = j % DNB
        pend[("d", bj)].wait()
        pend[("s", bj)] = pltpu.async_copy(
            ones_v, acc.at[dchunk.at[bj]], ssem[bj], add=True)
    for b in range(min(DNB, DNCHUNKS)):
        pend[("s", b)].wait()

    pltpu.sync_copy(dst_hbm.at[pl.ds(base + DNCHUNKS * DCHUNK, DTAIL)], dtail)
    pltpu.sync_copy(ones_v.at[pl.ds(0, DTAIL)], acc.at[dtail], add=True)

    plsc.subcore_barrier()

    @pl.when(s < 10)
    def _():
        pltpu.sync_copy(acc.at[pl.ds(s * WB_ROWS, WB_ROWS)],
                        zbuf.at[pl.ds(0, WB_ROWS)])
        pltpu.sync_copy(zbuf.at[pl.ds(0, WB_ROWS)],
                        out_hbm.at[pl.ds(c * N + s * WB_ROWS, WB_ROWS)])


NB = 2


@functools.partial(
    pl.kernel, mesh=_mesh,
    out_type=(jax.ShapeDtypeStruct((N, D), jnp.float32),
              jax.ShapeDtypeStruct((N, D), jnp.float32)),
    scratch_types=[
        pltpu.VMEM((E_PER_TILE,), jnp.int32),
        pltpu.VMEM((NB, CHUNK), jnp.int32),
        pltpu.VMEM((TAIL,), jnp.int32),
        pltpu.VMEM((NB, CHUNK, D), jnp.float32),
        pltpu.VMEM((ZROWS, D), jnp.float32),
        pltpu.VMEM_SHARED((N, D), jnp.float32),
        pltpu.SemaphoreType.DMA,
        pltpu.SemaphoreType.DMA,
        pltpu.SemaphoreType.DMA,
        pltpu.SemaphoreType.DMA,
        pltpu.SemaphoreType.DMA,
        pltpu.SemaphoreType.DMA,
    ])
def _sc_agg(hp_hbm, src_hbm, dst_hbm, out0_hbm, out1_hbm,
            src_v, dchunk, dtail, rows_v, zbuf, acc, g0, g1, s0, s1, d0, d1):
    c = lax.axis_index("c")
    s = lax.axis_index("s")
    t = c * NS + s
    base = t * E_PER_TILE
    gsem = (g0, g1)
    ssem = (s0, s1)
    dsem = (d0, d1)
    pltpu.sync_copy(src_hbm.at[pl.ds(base, E_PER_TILE)], src_v)

    z16 = jnp.zeros((16,), jnp.float32)

    @pl.loop(0, ZROWS)
    def _(r):
        @pl.loop(0, D, step=16)
        def _(cc):
            zbuf[r, pl.ds(cc, 16)] = z16

    @pl.when(s < 10)
    def _():
        @pl.loop(0, WB_ROWS // ZROWS)
        def _(i):
            pltpu.sync_copy(zbuf, acc.at[pl.ds(s * WB_ROWS + i * ZROWS, ZROWS)])

    plsc.subcore_barrier()

    pend = {}
    for k in range(NCHUNKS):
        b = k % NB
        if k >= NB:
            pend[("s", b)].wait()
        pend[("d", b)] = pltpu.async_copy(
            dst_hbm.at[pl.ds(base + k * CHUNK, CHUNK)], dchunk.at[b], dsem[b])
        pend[("g", b)] = pltpu.async_copy(
            hp_hbm.at[src_v.at[pl.ds(k * CHUNK, CHUNK)]], rows_v.at[b],
            gsem[b])
        j = k - (NB - 1)
        if j >= 0:
            bj = j % NB
            pend[("d", bj)].wait()
            pend[("g", bj)].wait()
            pend[("s", bj)] = pltpu.async_copy(
                rows_v.at[bj], acc.at[dchunk.at[bj]], ssem[bj], add=True)
    for j in range(max(0, NCHUNKS - NB + 1), NCHUNKS):
        bj = j % NB
        pend[("d", bj)].wait()
        pend[("g", bj)].wait()
        pend[("s", bj)] = pltpu.async_copy(
            rows_v.at[bj], acc.at[dchunk.at[bj]], ssem[bj], add=True)
    for b in range(min(NB, NCHUNKS)):
        pend[("s", b)].wait()

    pltpu.sync_copy(dst_hbm.at[pl.ds(base + NCHUNKS * CHUNK, TAIL)], dtail)
    pltpu.sync_copy(hp_hbm.at[src_v.at[pl.ds(NCHUNKS * CHUNK, TAIL)]],
                    rows_v.at[0, pl.ds(0, TAIL)])
    pltpu.sync_copy(rows_v.at[0, pl.ds(0, TAIL)], acc.at[dtail], add=True)

    plsc.subcore_barrier()

    @pl.when(s < 10)
    def _():
        @pl.loop(0, WB_ROWS // ZROWS)
        def _(i):
            pltpu.sync_copy(acc.at[pl.ds(s * WB_ROWS + i * ZROWS, ZROWS)],
                            zbuf)

            @pl.when(c == 0)
            def _():
                pltpu.sync_copy(
                    zbuf,
                    out0_hbm.at[pl.ds(s * WB_ROWS + i * ZROWS, ZROWS)])

            @pl.when(c == 1)
            def _():
                pltpu.sync_copy(
                    zbuf,
                    out1_hbm.at[pl.ds(s * WB_ROWS + i * ZROWS, ZROWS)])


BR = 2000
GRID = N // BR

def _rows(i):
    return (i, 0)

def _full(i):
    return (0, 0)


def _tc_scale(deg3, x, w1):
    def body(deg_ref, x_ref, w_ref, dis_ref, hp_ref):
        dis = lax.rsqrt(deg_ref[0] + deg_ref[1] + 1.0)
        dis_ref[...] = dis
        u = lax.dot_general(x_ref[...], w_ref[...],
                            (((1,), (0,)), ((), ())), **MM_KW)
        hp_ref[...] = dis * u
    return pl.pallas_call(
        body, grid=(GRID,),
        in_specs=[pl.BlockSpec((2, BR, 1), lambda i: (0, i, 0)),
                  pl.BlockSpec((BR, D), _rows), pl.BlockSpec((D, D), _full)],
        out_specs=(pl.BlockSpec((BR, 1), _rows), pl.BlockSpec((BR, D), _rows)),
        out_shape=(jax.ShapeDtypeStruct((N, 1), jnp.float32),
                   jax.ShapeDtypeStruct((N, D), jnp.float32)))(deg3, x, w1)


def _tc_layer(a0, a1, hp, dis, b, g, bt, m, v, w_next):
    def body(a0_ref, a1_ref, hp_ref, dis_ref, b_ref, g_ref, bt_ref, m_ref,
             v_ref, w_ref, o_ref):
        dis = dis_ref[...]
        z = (a0_ref[...] + a1_ref[...] + hp_ref[...]) * dis + b_ref[...]
        sc = g_ref[...] * lax.rsqrt(v_ref[...] + EPS)
        sh = bt_ref[...] - m_ref[...] * sc
        h = jnp.maximum(z * sc + sh, 0.0)
        o_ref[...] = dis * lax.dot_general(h, w_ref[...],
                                           (((1,), (0,)), ((), ())), **MM_KW)
    vec = pl.BlockSpec((1, D), _full)
    return pl.pallas_call(
        body, grid=(GRID,),
        in_specs=[pl.BlockSpec((BR, D), _rows), pl.BlockSpec((BR, D), _rows),
                  pl.BlockSpec((BR, D), _rows), pl.BlockSpec((BR, 1), _rows),
                  vec, vec, vec, vec, vec, pl.BlockSpec((D, D), _full)],
        out_specs=pl.BlockSpec((BR, D), _rows),
        out_shape=jax.ShapeDtypeStruct((N, D), jnp.float32))(
            a0, a1, hp, dis, b, g, bt, m, v, w_next)


def _tc_layer_now(a0, a1, hp, dis, b, g, bt, m, v):
    def body(a0_ref, a1_ref, hp_ref, dis_ref, b_ref, g_ref, bt_ref, m_ref,
             v_ref, o_ref):
        dis = dis_ref[...]
        z = (a0_ref[...] + a1_ref[...] + hp_ref[...]) * dis + b_ref[...]
        sc = g_ref[...] * lax.rsqrt(v_ref[...] + EPS)
        sh = bt_ref[...] - m_ref[...] * sc
        o_ref[...] = dis * jnp.maximum(z * sc + sh, 0.0)
    vec = pl.BlockSpec((1, D), _full)
    return pl.pallas_call(
        body, grid=(GRID,),
        in_specs=[pl.BlockSpec((BR, D), _rows), pl.BlockSpec((BR, D), _rows),
                  pl.BlockSpec((BR, D), _rows), pl.BlockSpec((BR, 1), _rows),
                  vec, vec, vec, vec, vec],
        out_specs=pl.BlockSpec((BR, D), _rows),
        out_shape=jax.ShapeDtypeStruct((N, D), jnp.float32))(
            a0, a1, hp, dis, b, g, bt, m, v)


def _tc_final(a0, a1, hph, dis, w3, b3):
    def body(a0_ref, a1_ref, hp_ref, dis_ref, w_ref, b_ref, o_ref):
        z = (a0_ref[...] + a1_ref[...] + hp_ref[...]) * dis_ref[...]
        o = lax.dot_general(z, w_ref[...],
                            (((1,), (0,)), ((), ())), **MM_KW) + b_ref[...]
        mx = jnp.max(o, axis=1, keepdims=True)
        lse = jnp.log(jnp.sum(jnp.exp(o - mx), axis=1, keepdims=True))
        o_ref[...] = o - mx - lse
    return pl.pallas_call(
        body, grid=(GRID,),
        in_specs=[pl.BlockSpec((BR, D), _rows), pl.BlockSpec((BR, D), _rows),
                  pl.BlockSpec((BR, D), _rows), pl.BlockSpec((BR, 1), _rows),
                  pl.BlockSpec((D, OUT), _full), pl.BlockSpec((1, OUT), _full)],
        out_specs=pl.BlockSpec((BR, OUT), _rows),
        out_shape=jax.ShapeDtypeStruct((N, OUT), jnp.float32))(
            a0, a1, hph, dis, w3, b3)


def kernel(x, edge_index, W1, b1, W2, b2, W3, b3,
           g1, bt1, m1, v1, g2, bt2, m2, v2):
    src = edge_index[0]
    dst = edge_index[1]

    degp = _sc_degree(dst)
    deg3 = degp.reshape(NC, N, 1)
    dis, hp1 = _tc_scale(deg3, x, W1)

    a0, a1 = _sc_agg(hp1, src, dst)
    hp2 = _tc_layer(a0, a1, hp1, dis, b1.reshape(1, D),
                    g1.reshape(1, D), bt1.reshape(1, D),
                    m1.reshape(1, D), v1.reshape(1, D), W2)

    a0, a1 = _sc_agg(hp2, src, dst)
    hph2 = _tc_layer_now(a0, a1, hp2, dis, b2.reshape(1, D),
                         g2.reshape(1, D), bt2.reshape(1, D),
                         m2.reshape(1, D), v2.reshape(1, D))

    a0, a1 = _sc_agg(hph2, src, dst)
    return _tc_final(a0, a1, hph2, dis, W3, b3.reshape(1, OUT))


# speedup vs baseline: 1.0024x; 1.0024x over previous
import functools

import jax
import jax.numpy as jnp
from jax import lax
from jax.experimental import pallas as pl
from jax.experimental.pallas import tpu as pltpu
from jax.experimental.pallas import tpu_sc as plsc

N = 10000
E = 320000
D = 128
OUT = 40
EPS = 1e-5

NC = 2
NS = 16
E_PER_TILE = E // (NC * NS)
CHUNK = 128
NCHUNKS = E_PER_TILE // CHUNK
TAIL = E_PER_TILE - NCHUNKS * CHUNK
ZROWS = 40
WB_ROWS = 1000

_mesh = plsc.VectorSubcoreMesh(core_axis_name="c", subcore_axis_name="s")

MM_KW = dict(preferred_element_type=jnp.float32,
             precision=jax.lax.Precision.HIGHEST)


DCHUNK = 128
DNCHUNKS = E_PER_TILE // DCHUNK
DTAIL = E_PER_TILE - DNCHUNKS * DCHUNK
DNB = 2


@functools.partial(
    pl.kernel, mesh=_mesh,
    out_type=jax.ShapeDtypeStruct((NC * N,), jnp.float32),
    scratch_types=[
        pltpu.VMEM((DNB, DCHUNK), jnp.int32),
        pltpu.VMEM((DTAIL,), jnp.int32),
        pltpu.VMEM((DCHUNK,), jnp.float32),
        pltpu.VMEM((1024,), jnp.float32),
        pltpu.VMEM_SHARED((10240,), jnp.float32),
        pltpu.SemaphoreType.DMA,
        pltpu.SemaphoreType.DMA,
        pltpu.SemaphoreType.DMA,
        pltpu.SemaphoreType.DMA,
    ])
def _sc_degree(dst_hbm, out_hbm, dchunk, dtail, ones_v, zbuf, acc,
               d0, d1, s0, s1):
    c = lax.axis_index("c")
    s = lax.axis_index("s")
    t = c * NS + s
    base = t * E_PER_TILE
    dsem = (d0, d1)
    ssem = (s0, s1)
    z16 = jnp.zeros((16,), jnp.float32)
    o16 = jnp.ones((16,), jnp.float32)

    @pl.loop(0, DCHUNK, step=16)
    def _(i):
        ones_v[pl.ds(i, 16)] = o16

    @pl.loop(0, 1024, step=16)
    def _(i):
        zbuf[pl.ds(i, 16)] = z16

    @pl.when(s < 10)
    def _():
        pltpu.sync_copy(zbuf, acc.at[pl.ds(s * 1024, 1024)])

    plsc.subcore_barrier()

    pend = {}
    for k in range(DNCHUNKS):
        b = k % DNB
        if k >= DNB:
            pend[("s", b)].wait()
        pend[("d", b)] = pltpu.async_copy(
            dst_hbm.at[pl.ds(base + k * DCHUNK, DCHUNK)], dchunk.at[b],
            dsem[b])
        j = k - (DNB - 1)
        if j >= 0:
            bj = j % DNB
            pend[("d", bj)].wait()
            pend[("s", bj)] = pltpu.async_copy(
                ones_v, acc.at[dchunk.at[bj]], ssem[bj], add=True)
    for j in range(max(0, DNCHUNKS - DNB + 1), DNCHUNKS):
        bj = j % DNB
        pend[("d", bj)].wait()
        pend[("s", bj)] = pltpu.async_copy(
            ones_v, acc.at[dchunk.at[bj]], ssem[bj], add=True)
    for b in range(min(DNB, DNCHUNKS)):
        pend[("s", b)].wait()

    pltpu.sync_copy(dst_hbm.at[pl.ds(base + DNCHUNKS * DCHUNK, DTAIL)], dtail)
    pltpu.sync_copy(ones_v.at[pl.ds(0, DTAIL)], acc.at[dtail], add=True)

    plsc.subcore_barrier()

    @pl.when(s < 10)
    def _():
        pltpu.sync_copy(acc.at[pl.ds(s * WB_ROWS, WB_ROWS)],
                        zbuf.at[pl.ds(0, WB_ROWS)])
        pltpu.sync_copy(zbuf.at[pl.ds(0, WB_ROWS)],
                        out_hbm.at[pl.ds(c * N + s * WB_ROWS, WB_ROWS)])


NB = 2


@functools.partial(
    pl.kernel, mesh=_mesh,
    out_type=(jax.ShapeDtypeStruct((N, D), jnp.float32),
              jax.ShapeDtypeStruct((N, D), jnp.float32)),
    scratch_types=[
        pltpu.VMEM((E_PER_TILE,), jnp.int32),
        pltpu.VMEM((NB, CHUNK), jnp.int32),
        pltpu.VMEM((TAIL,), jnp.int32),
        pltpu.VMEM((NB, CHUNK, D), jnp.float32),
        pltpu.VMEM((ZROWS, D), jnp.float32),
        pltpu.VMEM_SHARED((N, D), jnp.float32),
        pltpu.SemaphoreType.DMA,
        pltpu.SemaphoreType.DMA,
        pltpu.SemaphoreType.DMA,
        pltpu.SemaphoreType.DMA,
        pltpu.SemaphoreType.DMA,
        pltpu.SemaphoreType.DMA,
    ])
def _sc_agg(hp_hbm, src_hbm, dst_hbm, out0_hbm, out1_hbm,
            src_v, dchunk, dtail, rows_v, zbuf, acc, g0, g1, s0, s1, d0, d1):
    c = lax.axis_index("c")
    s = lax.axis_index("s")
    t = c * NS + s
    base = t * E_PER_TILE
    gsem = (g0, g1)
    ssem = (s0, s1)
    dsem = (d0, d1)
    pltpu.sync_copy(src_hbm.at[pl.ds(base, E_PER_TILE)], src_v)

    z16 = jnp.zeros((16,), jnp.float32)

    @pl.loop(0, ZROWS)
    def _(r):
        @pl.loop(0, D, step=16)
        def _(cc):
            zbuf[r, pl.ds(cc, 16)] = z16

    @pl.when(s < 10)
    def _():
        @pl.loop(0, WB_ROWS // ZROWS)
        def _(i):
            pltpu.sync_copy(zbuf, acc.at[pl.ds(s * WB_ROWS + i * ZROWS, ZROWS)])

    plsc.subcore_barrier()

    pend = {}
    for k in range(NCHUNKS):
        b = k % NB
        if k >= NB:
            pend[("s", b)].wait()
        pend[("d", b)] = pltpu.async_copy(
            dst_hbm.at[pl.ds(base + k * CHUNK, CHUNK)], dchunk.at[b], dsem[b])
        pend[("g", b)] = pltpu.async_copy(
            hp_hbm.at[src_v.at[pl.ds(k * CHUNK, CHUNK)]], rows_v.at[b],
            gsem[b])
        j = k - (NB - 1)
        if j >= 0:
            bj = j % NB
            pend[("d", bj)].wait()
            pend[("g", bj)].wait()
            pend[("s", bj)] = pltpu.async_copy(
                rows_v.at[bj], acc.at[dchunk.at[bj]], ssem[bj], add=True)
    for j in range(max(0, NCHUNKS - NB + 1), NCHUNKS):
        bj = j % NB
        pend[("d", bj)].wait()
        pend[("g", bj)].wait()
        pend[("s", bj)] = pltpu.async_copy(
            rows_v.at[bj], acc.at[dchunk.at[bj]], ssem[bj], add=True)
    for b in range(min(NB, NCHUNKS)):
        pend[("s", b)].wait()

    pltpu.sync_copy(dst_hbm.at[pl.ds(base + NCHUNKS * CHUNK, TAIL)], dtail)
    pltpu.sync_copy(hp_hbm.at[src_v.at[pl.ds(NCHUNKS * CHUNK, TAIL)]],
                    rows_v.at[0, pl.ds(0, TAIL)])
    pltpu.sync_copy(rows_v.at[0, pl.ds(0, TAIL)], acc.at[dtail], add=True)

    plsc.subcore_barrier()

    @pl.when(s < 10)
    def _():
        @pl.loop(0, WB_ROWS // ZROWS)
        def _(i):
            pltpu.sync_copy(acc.at[pl.ds(s * WB_ROWS + i * ZROWS, ZROWS)],
                            zbuf)

            @pl.when(c == 0)
            def _():
                pltpu.sync_copy(
                    zbuf,
                    out0_hbm.at[pl.ds(s * WB_ROWS + i * ZROWS, ZROWS)])

            @pl.when(c == 1)
            def _():
                pltpu.sync_copy(
                    zbuf,
                    out1_hbm.at[pl.ds(s * WB_ROWS + i * ZROWS, ZROWS)])


BR = 2000
GRID = N // BR

def _rows(i):
    return (i, 0)

def _full(i):
    return (0, 0)


def _tc_mm(x, w):
    def body(x_ref, w_ref, o_ref):
        o_ref[...] = lax.dot_general(x_ref[...], w_ref[...],
                                     (((1,), (0,)), ((), ())), **MM_KW)
    return pl.pallas_call(
        body, grid=(GRID,),
        in_specs=[pl.BlockSpec((BR, D), _rows), pl.BlockSpec((D, D), _full)],
        out_specs=pl.BlockSpec((BR, D), _rows),
        out_shape=jax.ShapeDtypeStruct((N, D), jnp.float32))(x, w)


def _tc_scale(deg3, u1):
    def body(deg_ref, u_ref, dis_ref, hp_ref):
        dis = lax.rsqrt(deg_ref[0] + deg_ref[1] + 1.0)
        dis_ref[...] = dis
        hp_ref[...] = dis * u_ref[...]
    return pl.pallas_call(
        body, grid=(GRID,),
        in_specs=[pl.BlockSpec((2, BR, 1), lambda i: (0, i, 0)),
                  pl.BlockSpec((BR, D), _rows)],
        out_specs=(pl.BlockSpec((BR, 1), _rows), pl.BlockSpec((BR, D), _rows)),
        out_shape=(jax.ShapeDtypeStruct((N, 1), jnp.float32),
                   jax.ShapeDtypeStruct((N, D), jnp.float32)))(deg3, u1)


def _tc_layer(a0, a1, hp, dis, b, g, bt, m, v, w_next):
    def body(a0_ref, a1_ref, hp_ref, dis_ref, b_ref, g_ref, bt_ref, m_ref,
             v_ref, w_ref, o_ref):
        dis = dis_ref[...]
        z = (a0_ref[...] + a1_ref[...] + hp_ref[...]) * dis + b_ref[...]
        sc = g_ref[...] * lax.rsqrt(v_ref[...] + EPS)
        sh = bt_ref[...] - m_ref[...] * sc
        h = jnp.maximum(z * sc + sh, 0.0)
        o_ref[...] = dis * lax.dot_general(h, w_ref[...],
                                           (((1,), (0,)), ((), ())), **MM_KW)
    vec = pl.BlockSpec((1, D), _full)
    return pl.pallas_call(
        body, grid=(GRID,),
        in_specs=[pl.BlockSpec((BR, D), _rows), pl.BlockSpec((BR, D), _rows),
                  pl.BlockSpec((BR, D), _rows), pl.BlockSpec((BR, 1), _rows),
                  vec, vec, vec, vec, vec, pl.BlockSpec((D, D), _full)],
        out_specs=pl.BlockSpec((BR, D), _rows),
        out_shape=jax.ShapeDtypeStruct((N, D), jnp.float32))(
            a0, a1, hp, dis, b, g, bt, m, v, w_next)


def _tc_layer_now(a0, a1, hp, dis, b, g, bt, m, v):
    def body(a0_ref, a1_ref, hp_ref, dis_ref, b_ref, g_ref, bt_ref, m_ref,
             v_ref, o_ref):
        dis = dis_ref[...]
        z = (a0_ref[...] + a1_ref[...] + hp_ref[...]) * dis + b_ref[...]
        sc = g_ref[...] * lax.rsqrt(v_ref[...] + EPS)
        sh = bt_ref[...] - m_ref[...] * sc
        o_ref[...] = dis * jnp.maximum(z * sc + sh, 0.0)
    vec = pl.BlockSpec((1, D), _full)
    return pl.pallas_call(
        body, grid=(GRID,),
        in_specs=[pl.BlockSpec((BR, D), _rows), pl.BlockSpec((BR, D), _rows),
                  pl.BlockSpec((BR, D), _rows), pl.BlockSpec((BR, 1), _rows),
                  vec, vec, vec, vec, vec],
        out_specs=pl.BlockSpec((BR, D), _rows),
        out_shape=jax.ShapeDtypeStruct((N, D), jnp.float32))(
            a0, a1, hp, dis, b, g, bt, m, v)


def _tc_final(a0, a1, hph, dis, w3, b3):
    def body(a0_ref, a1_ref, hp_ref, dis_ref, w_ref, b_ref, o_ref):
        z = (a0_ref[...] + a1_ref[...] + hp_ref[...]) * dis_ref[...]
        o = lax.dot_general(z, w_ref[...],
                            (((1,), (0,)), ((), ())), **MM_KW) + b_ref[...]
        mx = jnp.max(o, axis=1, keepdims=True)
        lse = jnp.log(jnp.sum(jnp.exp(o - mx), axis=1, keepdims=True))
        o_ref[...] = o - mx - lse
    return pl.pallas_call(
        body, grid=(GRID,),
        in_specs=[pl.BlockSpec((BR, D), _rows), pl.BlockSpec((BR, D), _rows),
                  pl.BlockSpec((BR, D), _rows), pl.BlockSpec((BR, 1), _rows),
                  pl.BlockSpec((D, OUT), _full), pl.BlockSpec((1, OUT), _full)],
        out_specs=pl.BlockSpec((BR, OUT), _rows),
        out_shape=jax.ShapeDtypeStruct((N, OUT), jnp.float32))(
            a0, a1, hph, dis, w3, b3)


def kernel(x, edge_index, W1, b1, W2, b2, W3, b3,
           g1, bt1, m1, v1, g2, bt2, m2, v2):
    src = edge_index[0]
    dst = edge_index[1]

    degp = _sc_degree(dst)
    u1 = _tc_mm(x, W1)
    deg3 = degp.reshape(NC, N, 1)
    dis, hp1 = _tc_scale(deg3, u1)

    a0, a1 = _sc_agg(hp1, src, dst)
    hp2 = _tc_layer(a0, a1, hp1, dis, b1.reshape(1, D),
                    g1.reshape(1, D), bt1.reshape(1, D),
                    m1.reshape(1, D), v1.reshape(1, D), W2)

    a0, a1 = _sc_agg(hp2, src, dst)
    hph2 = _tc_layer_now(a0, a1, hp2, dis, b2.reshape(1, D),
                         g2.reshape(1, D), bt2.reshape(1, D),
                         m2.reshape(1, D), v2.reshape(1, D))

    a0, a1 = _sc_agg(hph2, src, dst)
    return _tc_final(a0, a1, hph2, dis, W3, b3.reshape(1, OUT))


# speedup vs baseline: 1.0240x; 1.0215x over previous
import functools

import jax
import jax.numpy as jnp
from jax import lax
from jax.experimental import pallas as pl
from jax.experimental.pallas import tpu as pltpu
from jax.experimental.pallas import tpu_sc as plsc

N = 10000
E = 320000
D = 128
OUT = 40
EPS = 1e-5

NC = 2
NS = 16
E_PER_TILE = E // (NC * NS)
CHUNK = 128
NCHUNKS = E_PER_TILE // CHUNK
TAIL = E_PER_TILE - NCHUNKS * CHUNK
ZROWS = 40
WB_ROWS = 1000

_mesh = plsc.VectorSubcoreMesh(core_axis_name="c", subcore_axis_name="s")

MM_KW = dict(preferred_element_type=jnp.float32,
             precision=jax.lax.Precision.HIGHEST)


DCHUNK = 80
DNCHUNKS = E_PER_TILE // DCHUNK


@functools.partial(
    pl.kernel, mesh=_mesh,
    out_type=jax.ShapeDtypeStruct((NC * N,), jnp.float32),
    scratch_types=[
        pltpu.VMEM((E_PER_TILE,), jnp.int32),
        pltpu.VMEM((DCHUNK,), jnp.int32),
        pltpu.VMEM((DCHUNK,), jnp.float32),
        pltpu.VMEM((1024,), jnp.float32),
        pltpu.VMEM_SHARED((10240,), jnp.float32),
        pltpu.SemaphoreType.DMA,
    ])
def _sc_degree(dst_hbm, out_hbm, dst_v, dchunk, ones_v, zbuf, acc, sem):
    c = lax.axis_index("c")
    s = lax.axis_index("s")
    t = c * NS + s
    pltpu.sync_copy(dst_hbm.at[pl.ds(t * E_PER_TILE, E_PER_TILE)], dst_v)
    z16 = jnp.zeros((16,), jnp.float32)
    o16 = jnp.ones((16,), jnp.float32)

    @pl.loop(0, DCHUNK, step=16)
    def _(i):
        ones_v[pl.ds(i, 16)] = o16

    @pl.loop(0, 1024, step=16)
    def _(i):
        zbuf[pl.ds(i, 16)] = z16

    @pl.when(s < 10)
    def _():
        pltpu.sync_copy(zbuf, acc.at[pl.ds(s * 1024, 1024)])

    plsc.subcore_barrier()

    @pl.loop(0, DNCHUNKS)
    def _(k):
        @pl.loop(0, DCHUNK, step=16)
        def _(i):
            dchunk[pl.ds(i, 16)] = dst_v[pl.ds(k * DCHUNK + i, 16)]
        pltpu.sync_copy(ones_v, acc.at[dchunk], add=True)

    plsc.subcore_barrier()

    @pl.when(s < 10)
    def _():
        pltpu.sync_copy(acc.at[pl.ds(s * WB_ROWS, WB_ROWS)],
                        zbuf.at[pl.ds(0, WB_ROWS)])
        pltpu.sync_copy(zbuf.at[pl.ds(0, WB_ROWS)],
                        out_hbm.at[pl.ds(c * N + s * WB_ROWS, WB_ROWS)])


NB = 2


@functools.partial(
    pl.kernel, mesh=_mesh,
    out_type=(jax.ShapeDtypeStruct((N, D), jnp.float32),
              jax.ShapeDtypeStruct((N, D), jnp.float32)),
    scratch_types=[
        pltpu.VMEM((E_PER_TILE,), jnp.int32),
        pltpu.VMEM((NB, CHUNK), jnp.int32),
        pltpu.VMEM((TAIL,), jnp.int32),
        pltpu.VMEM((NB, CHUNK, D), jnp.float32),
        pltpu.VMEM((ZROWS, D), jnp.float32),
        pltpu.VMEM_SHARED((N, D), jnp.float32),
        pltpu.SemaphoreType.DMA,
        pltpu.SemaphoreType.DMA,
        pltpu.SemaphoreType.DMA,
        pltpu.SemaphoreType.DMA,
        pltpu.SemaphoreType.DMA,
        pltpu.SemaphoreType.DMA,
    ])
def _sc_agg(hp_hbm, src_hbm, dst_hbm, out0_hbm, out1_hbm,
            src_v, dchunk, dtail, rows_v, zbuf, acc, g0, g1, s0, s1, d0, d1):
    c = lax.axis_index("c")
    s = lax.axis_index("s")
    t = c * NS + s
    base = t * E_PER_TILE
    gsem = (g0, g1)
    ssem = (s0, s1)
    dsem = (d0, d1)
    pltpu.sync_copy(src_hbm.at[pl.ds(base, E_PER_TILE)], src_v)

    z16 = jnp.zeros((16,), jnp.float32)

    @pl.loop(0, ZROWS)
    def _(r):
        @pl.loop(0, D, step=16)
        def _(cc):
            zbuf[r, pl.ds(cc, 16)] = z16

    @pl.when(s < 10)
    def _():
        @pl.loop(0, WB_ROWS // ZROWS)
        def _(i):
            pltpu.sync_copy(zbuf, acc.at[pl.ds(s * WB_ROWS + i * ZROWS, ZROWS)])

    plsc.subcore_barrier()

    pend = {}
    for k in range(NCHUNKS):
        b = k % NB
        if k >= NB:
            pend[("s", b)].wait()
        pend[("d", b)] = pltpu.async_copy(
            dst_hbm.at[pl.ds(base + k * CHUNK, CHUNK)], dchunk.at[b], dsem[b])
        pend[("g", b)] = pltpu.async_copy(
            hp_hbm.at[src_v.at[pl.ds(k * CHUNK, CHUNK)]], rows_v.at[b],
            gsem[b])
        j = k - (NB - 1)
        if j >= 0:
            bj = j % NB
            pend[("d", bj)].wait()
            pend[("g", bj)].wait()
            pend[("s", bj)] = pltpu.async_copy(
                rows_v.at[bj], acc.at[dchunk.at[bj]], ssem[bj], add=True)
    for j in range(max(0, NCHUNKS - NB + 1), NCHUNKS):
        bj = j % NB
        pend[("d", bj)].wait()
        pend[("g", bj)].wait()
        pend[("s", bj)] = pltpu.async_copy(
            rows_v.at[bj], acc.at[dchunk.at[bj]], ssem[bj], add=True)
    for b in range(min(NB, NCHUNKS)):
        pend[("s", b)].wait()

    pltpu.sync_copy(dst_hbm.at[pl.ds(base + NCHUNKS * CHUNK, TAIL)], dtail)
    pltpu.sync_copy(hp_hbm.at[src_v.at[pl.ds(NCHUNKS * CHUNK, TAIL)]],
                    rows_v.at[0, pl.ds(0, TAIL)])
    pltpu.sync_copy(rows_v.at[0, pl.ds(0, TAIL)], acc.at[dtail], add=True)

    plsc.subcore_barrier()

    @pl.when(s < 10)
    def _():
        @pl.loop(0, WB_ROWS // ZROWS)
        def _(i):
            pltpu.sync_copy(acc.at[pl.ds(s * WB_ROWS + i * ZROWS, ZROWS)],
                            zbuf)

            @pl.when(c == 0)
            def _():
                pltpu.sync_copy(
                    zbuf,
                    out0_hbm.at[pl.ds(s * WB_ROWS + i * ZROWS, ZROWS)])

            @pl.when(c == 1)
            def _():
                pltpu.sync_copy(
                    zbuf,
                    out1_hbm.at[pl.ds(s * WB_ROWS + i * ZROWS, ZROWS)])


BR = 2000
GRID = N // BR

def _rows(i):
    return (i, 0)

def _full(i):
    return (0, 0)


def _tc_mm(x, w):
    def body(x_ref, w_ref, o_ref):
        o_ref[...] = lax.dot_general(x_ref[...], w_ref[...],
                                     (((1,), (0,)), ((), ())), **MM_KW)
    return pl.pallas_call(
        body, grid=(GRID,),
        in_specs=[pl.BlockSpec((BR, D), _rows), pl.BlockSpec((D, D), _full)],
        out_specs=pl.BlockSpec((BR, D), _rows),
        out_shape=jax.ShapeDtypeStruct((N, D), jnp.float32))(x, w)


def _tc_scale(deg3, u1):
    def body(deg_ref, u_ref, dis_ref, hp_ref):
        dis = lax.rsqrt(deg_ref[0] + deg_ref[1] + 1.0)
        dis_ref[...] = dis
        hp_ref[...] = dis * u_ref[...]
    return pl.pallas_call(
        body, grid=(GRID,),
        in_specs=[pl.BlockSpec((2, BR, 1), lambda i: (0, i, 0)),
                  pl.BlockSpec((BR, D), _rows)],
        out_specs=(pl.BlockSpec((BR, 1), _rows), pl.BlockSpec((BR, D), _rows)),
        out_shape=(jax.ShapeDtypeStruct((N, 1), jnp.float32),
                   jax.ShapeDtypeStruct((N, D), jnp.float32)))(deg3, u1)


def _tc_layer(a0, a1, hp, dis, b, g, bt, m, v, w_next):
    def body(a0_ref, a1_ref, hp_ref, dis_ref, b_ref, g_ref, bt_ref, m_ref,
             v_ref, w_ref, o_ref):
        dis = dis_ref[...]
        z = (a0_ref[...] + a1_ref[...] + hp_ref[...]) * dis + b_ref[...]
        sc = g_ref[...] * lax.rsqrt(v_ref[...] + EPS)
        sh = bt_ref[...] - m_ref[...] * sc
        h = jnp.maximum(z * sc + sh, 0.0)
        o_ref[...] = dis * lax.dot_general(h, w_ref[...],
                                           (((1,), (0,)), ((), ())), **MM_KW)
    vec = pl.BlockSpec((1, D), _full)
    return pl.pallas_call(
        body, grid=(GRID,),
        in_specs=[pl.BlockSpec((BR, D), _rows), pl.BlockSpec((BR, D), _rows),
                  pl.BlockSpec((BR, D), _rows), pl.BlockSpec((BR, 1), _rows),
                  vec, vec, vec, vec, vec, pl.BlockSpec((D, D), _full)],
        out_specs=pl.BlockSpec((BR, D), _rows),
        out_shape=jax.ShapeDtypeStruct((N, D), jnp.float32))(
            a0, a1, hp, dis, b, g, bt, m, v, w_next)


def _tc_layer_now(a0, a1, hp, dis, b, g, bt, m, v):
    def body(a0_ref, a1_ref, hp_ref, dis_ref, b_ref, g_ref, bt_ref, m_ref,
             v_ref, o_ref):
        dis = dis_ref[...]
        z = (a0_ref[...] + a1_ref[...] + hp_ref[...]) * dis + b_ref[...]
        sc = g_ref[...] * lax.rsqrt(v_ref[...] + EPS)
        sh = bt_ref[...] - m_ref[...] * sc
        o_ref[...] = dis * jnp.maximum(z * sc + sh, 0.0)
    vec = pl.BlockSpec((1, D), _full)
    return pl.pallas_call(
        body, grid=(GRID,),
        in_specs=[pl.BlockSpec((BR, D), _rows), pl.BlockSpec((BR, D), _rows),
                  pl.BlockSpec((BR, D), _rows), pl.BlockSpec((BR, 1), _rows),
                  vec, vec, vec, vec, vec],
        out_specs=pl.BlockSpec((BR, D), _rows),
        out_shape=jax.ShapeDtypeStruct((N, D), jnp.float32))(
            a0, a1, hp, dis, b, g, bt, m, v)


def _tc_final(a0, a1, hph, dis, w3, b3):
    def body(a0_ref, a1_ref, hp_ref, dis_ref, w_ref, b_ref, o_ref):
        z = (a0_ref[...] + a1_ref[...] + hp_ref[...]) * dis_ref[...]
        o = lax.dot_general(z, w_ref[...],
                            (((1,), (0,)), ((), ())), **MM_KW) + b_ref[...]
        mx = jnp.max(o, axis=1, keepdims=True)
        lse = jnp.log(jnp.sum(jnp.exp(o - mx), axis=1, keepdims=True))
        o_ref[...] = o - mx - lse
    return pl.pallas_call(
        body, grid=(GRID,),
        in_specs=[pl.BlockSpec((BR, D), _rows), pl.BlockSpec((BR, D), _rows),
                  pl.BlockSpec((BR, D), _rows), pl.BlockSpec((BR, 1), _rows),
                  pl.BlockSpec((D, OUT), _full), pl.BlockSpec((1, OUT), _full)],
        out_specs=pl.BlockSpec((BR, OUT), _rows),
        out_shape=jax.ShapeDtypeStruct((N, OUT), jnp.float32))(
            a0, a1, hph, dis, w3, b3)


def kernel(x, edge_index, W1, b1, W2, b2, W3, b3,
           g1, bt1, m1, v1, g2, bt2, m2, v2):
    src = edge_index[0]
    dst = edge_index[1]

    degp = _sc_degree(dst)
    u1 = _tc_mm(x, W1)
    deg3 = degp.reshape(NC, N, 1)
    dis, hp1 = _tc_scale(deg3, u1)

    a0, a1 = _sc_agg(hp1, src, dst)
    hp2 = _tc_layer(a0, a1, hp1, dis, b1.reshape(1, D),
                    g1.reshape(1, D), bt1.reshape(1, D),
                    m1.reshape(1, D), v1.reshape(1, D), W2)

    a0, a1 = _sc_agg(hp2, src, dst)
    hph2 = _tc_layer_now(a0, a1, hp2, dis, b2.reshape(1, D),
                         g2.reshape(1, D), bt2.reshape(1, D),
                         m2.reshape(1, D), v2.reshape(1, D))

    a0, a1 = _sc_agg(hph2, src, dst)
    return _tc_final(a0, a1, hph2, dis, W3, b3.reshape(1, OUT))


# speedup vs baseline: 1.0437x; 1.0192x over previous
import functools

import jax
import jax.numpy as jnp
from jax import lax
from jax.experimental import pallas as pl
from jax.experimental.pallas import tpu as pltpu
from jax.experimental.pallas import tpu_sc as plsc

N = 10000
E = 320000
D = 128
OUT = 40
EPS = 1e-5

NC = 2
NS = 16
E_PER_TILE = E // (NC * NS)
CHUNK = 128
NCHUNKS = E_PER_TILE // CHUNK
TAIL = E_PER_TILE - NCHUNKS * CHUNK
ZROWS = 40
WB_ROWS = 1000

_mesh = plsc.VectorSubcoreMesh(core_axis_name="c", subcore_axis_name="s")

MM_KW = dict(preferred_element_type=jnp.float32,
             precision=jax.lax.Precision.HIGHEST)


DCHUNK = 80
DNCHUNKS = E_PER_TILE // DCHUNK


@functools.partial(
    pl.kernel, mesh=_mesh,
    out_type=jax.ShapeDtypeStruct((NC * N,), jnp.float32),
    scratch_types=[
        pltpu.VMEM((E_PER_TILE,), jnp.int32),
        pltpu.VMEM((DCHUNK,), jnp.int32),
        pltpu.VMEM((DCHUNK,), jnp.float32),
        pltpu.VMEM((1024,), jnp.float32),
        pltpu.VMEM_SHARED((10240,), jnp.float32),
        pltpu.SemaphoreType.DMA,
    ])
def _sc_degree(dst_hbm, out_hbm, dst_v, dchunk, ones_v, zbuf, acc, sem):
    c = lax.axis_index("c")
    s = lax.axis_index("s")
    t = c * NS + s
    pltpu.sync_copy(dst_hbm.at[pl.ds(t * E_PER_TILE, E_PER_TILE)], dst_v)
    z16 = jnp.zeros((16,), jnp.float32)
    o16 = jnp.ones((16,), jnp.float32)

    @pl.loop(0, DCHUNK, step=16)
    def _(i):
        ones_v[pl.ds(i, 16)] = o16

    @pl.loop(0, 1024, step=16)
    def _(i):
        zbuf[pl.ds(i, 16)] = z16

    @pl.when(s < 10)
    def _():
        pltpu.sync_copy(zbuf, acc.at[pl.ds(s * 1024, 1024)])

    plsc.subcore_barrier()

    @pl.loop(0, DNCHUNKS)
    def _(k):
        @pl.loop(0, DCHUNK, step=16)
        def _(i):
            dchunk[pl.ds(i, 16)] = dst_v[pl.ds(k * DCHUNK + i, 16)]
        pltpu.sync_copy(ones_v, acc.at[dchunk], add=True)

    plsc.subcore_barrier()

    @pl.when(s < 10)
    def _():
        pltpu.sync_copy(acc.at[pl.ds(s * WB_ROWS, WB_ROWS)],
                        zbuf.at[pl.ds(0, WB_ROWS)])
        pltpu.sync_copy(zbuf.at[pl.ds(0, WB_ROWS)],
                        out_hbm.at[pl.ds(c * N + s * WB_ROWS, WB_ROWS)])


NB = 3


@functools.partial(
    pl.kernel, mesh=_mesh,
    out_type=(jax.ShapeDtypeStruct((N, D), jnp.float32),
              jax.ShapeDtypeStruct((N, D), jnp.float32)),
    scratch_types=[
        pltpu.VMEM((NB, CHUNK), jnp.int32),
        pltpu.VMEM((NB, CHUNK), jnp.int32),
        pltpu.VMEM((TAIL,), jnp.int32),
        pltpu.VMEM((TAIL,), jnp.int32),
        pltpu.VMEM((NB, CHUNK, D), jnp.float32),
        pltpu.VMEM_SHARED((N, D), jnp.float32),
        pltpu.SemaphoreType.DMA,
        pltpu.SemaphoreType.DMA,
        pltpu.SemaphoreType.DMA,
        pltpu.SemaphoreType.DMA,
        pltpu.SemaphoreType.DMA,
        pltpu.SemaphoreType.DMA,
        pltpu.SemaphoreType.DMA,
        pltpu.SemaphoreType.DMA,
        pltpu.SemaphoreType.DMA,
        pltpu.SemaphoreType.DMA,
        pltpu.SemaphoreType.DMA,
        pltpu.SemaphoreType.DMA,
    ])
def _sc_agg(hp_hbm, src_hbm, dst_hbm, out0_hbm, out1_hbm,
            schunk, dchunk, stail, dtail, rows_v, acc,
            i0, i1, i2, d0, d1, d2, g0, g1, g2, s0, s1, s2):
    c = lax.axis_index("c")
    s = lax.axis_index("s")
    t = c * NS + s
    base = t * E_PER_TILE
    isem = (i0, i1, i2)
    dsem = (d0, d1, d2)
    gsem = (g0, g1, g2)
    ssem = (s0, s1, s2)

    z16 = jnp.zeros((16,), jnp.float32)

    @pl.loop(0, CHUNK)
    def _(r):
        @pl.loop(0, D, step=16)
        def _(cc):
            rows_v[0, r, pl.ds(cc, 16)] = z16

    @pl.when(s < 10)
    def _():
        @pl.loop(0, 7)
        def _(i):
            pltpu.sync_copy(rows_v.at[0],
                            acc.at[pl.ds(s * WB_ROWS + i * CHUNK, CHUNK)])
        pltpu.sync_copy(rows_v.at[0, pl.ds(0, 104)],
                        acc.at[pl.ds(s * WB_ROWS + 7 * CHUNK, 104)])

    plsc.subcore_barrier()

    pend = {}
    for k in range(NCHUNKS + 2):
        b = k % NB
        if k < NCHUNKS:
            if k >= NB:
                pend[("s", b)].wait()
            pend[("i", b)] = pltpu.async_copy(
                src_hbm.at[pl.ds(base + k * CHUNK, CHUNK)], schunk.at[b],
                isem[b])
            pend[("d", b)] = pltpu.async_copy(
                dst_hbm.at[pl.ds(base + k * CHUNK, CHUNK)], dchunk.at[b],
                dsem[b])
        j = k - 1
        if 0 <= j < NCHUNKS:
            bj = j % NB
            pend[("i", bj)].wait()
            pend[("g", bj)] = pltpu.async_copy(
                hp_hbm.at[schunk.at[bj]], rows_v.at[bj], gsem[bj])
        j2 = k - 2
        if 0 <= j2 < NCHUNKS:
            b2 = j2 % NB
            pend[("g", b2)].wait()
            pend[("d", b2)].wait()
            pend[("s", b2)] = pltpu.async_copy(
                rows_v.at[b2], acc.at[dchunk.at[b2]], ssem[b2], add=True)
    for j in range(max(0, NCHUNKS - NB), NCHUNKS):
        pend[("s", j % NB)].wait()

    pltpu.sync_copy(src_hbm.at[pl.ds(base + NCHUNKS * CHUNK, TAIL)], stail)
    pltpu.sync_copy(dst_hbm.at[pl.ds(base + NCHUNKS * CHUNK, TAIL)], dtail)
    pltpu.sync_copy(hp_hbm.at[stail], rows_v.at[0, pl.ds(0, TAIL)])
    pltpu.sync_copy(rows_v.at[0, pl.ds(0, TAIL)], acc.at[dtail], add=True)

    plsc.subcore_barrier()

    @pl.when(s < 10)
    def _():
        @pl.loop(0, 7)
        def _(i):
            pltpu.sync_copy(acc.at[pl.ds(s * WB_ROWS + i * CHUNK, CHUNK)],
                            rows_v.at[0])

            @pl.when(c == 0)
            def _():
                pltpu.sync_copy(
                    rows_v.at[0],
                    out0_hbm.at[pl.ds(s * WB_ROWS + i * CHUNK, CHUNK)])

            @pl.when(c == 1)
            def _():
                pltpu.sync_copy(
                    rows_v.at[0],
                    out1_hbm.at[pl.ds(s * WB_ROWS + i * CHUNK, CHUNK)])
        pltpu.sync_copy(acc.at[pl.ds(s * WB_ROWS + 7 * CHUNK, 104)],
                        rows_v.at[0, pl.ds(0, 104)])

        @pl.when(c == 0)
        def _():
            pltpu.sync_copy(rows_v.at[0, pl.ds(0, 104)],
                            out0_hbm.at[pl.ds(s * WB_ROWS + 7 * CHUNK, 104)])

        @pl.when(c == 1)
        def _():
            pltpu.sync_copy(rows_v.at[0, pl.ds(0, 104)],
                            out1_hbm.at[pl.ds(s * WB_ROWS + 7 * CHUNK, 104)])


BR = 2000
GRID = N // BR

def _rows(i):
    return (i, 0)

def _full(i):
    return (0, 0)


def _tc_mm(x, w):
    def body(x_ref, w_ref, o_ref):
        o_ref[...] = lax.dot_general(x_ref[...], w_ref[...],
                                     (((1,), (0,)), ((), ())), **MM_KW)
    return pl.pallas_call(
        body, grid=(GRID,),
        in_specs=[pl.BlockSpec((BR, D), _rows), pl.BlockSpec((D, D), _full)],
        out_specs=pl.BlockSpec((BR, D), _rows),
        out_shape=jax.ShapeDtypeStruct((N, D), jnp.float32))(x, w)


def _tc_scale(deg3, u1):
    def body(deg_ref, u_ref, dis_ref, hp_ref):
        dis = lax.rsqrt(deg_ref[0] + deg_ref[1] + 1.0)
        dis_ref[...] = dis
        hp_ref[...] = dis * u_ref[...]
    return pl.pallas_call(
        body, grid=(GRID,),
        in_specs=[pl.BlockSpec((2, BR, 1), lambda i: (0, i, 0)),
                  pl.BlockSpec((BR, D), _rows)],
        out_specs=(pl.BlockSpec((BR, 1), _rows), pl.BlockSpec((BR, D), _rows)),
        out_shape=(jax.ShapeDtypeStruct((N, 1), jnp.float32),
                   jax.ShapeDtypeStruct((N, D), jnp.float32)))(deg3, u1)


def _tc_layer(a0, a1, hp, dis, b, g, bt, m, v, w_next):
    def body(a0_ref, a1_ref, hp_ref, dis_ref, b_ref, g_ref, bt_ref, m_ref,
             v_ref, w_ref, o_ref):
        dis = dis_ref[...]
        z = (a0_ref[...] + a1_ref[...] + hp_ref[...]) * dis + b_ref[...]
        sc = g_ref[...] * lax.rsqrt(v_ref[...] + EPS)
        sh = bt_ref[...] - m_ref[...] * sc
        h = jnp.maximum(z * sc + sh, 0.0)
        o_ref[...] = dis * lax.dot_general(h, w_ref[...],
                                           (((1,), (0,)), ((), ())), **MM_KW)
    vec = pl.BlockSpec((1, D), _full)
    return pl.pallas_call(
        body, grid=(GRID,),
        in_specs=[pl.BlockSpec((BR, D), _rows), pl.BlockSpec((BR, D), _rows),
                  pl.BlockSpec((BR, D), _rows), pl.BlockSpec((BR, 1), _rows),
                  vec, vec, vec, vec, vec, pl.BlockSpec((D, D), _full)],
        out_specs=pl.BlockSpec((BR, D), _rows),
        out_shape=jax.ShapeDtypeStruct((N, D), jnp.float32))(
            a0, a1, hp, dis, b, g, bt, m, v, w_next)


def _tc_layer_now(a0, a1, hp, dis, b, g, bt, m, v):
    def body(a0_ref, a1_ref, hp_ref, dis_ref, b_ref, g_ref, bt_ref, m_ref,
             v_ref, o_ref):
        dis = dis_ref[...]
        z = (a0_ref[...] + a1_ref[...] + hp_ref[...]) * dis + b_ref[...]
        sc = g_ref[...] * lax.rsqrt(v_ref[...] + EPS)
        sh = bt_ref[...] - m_ref[...] * sc
        o_ref[...] = dis * jnp.maximum(z * sc + sh, 0.0)
    vec = pl.BlockSpec((1, D), _full)
    return pl.pallas_call(
        body, grid=(GRID,),
        in_specs=[pl.BlockSpec((BR, D), _rows), pl.BlockSpec((BR, D), _rows),
                  pl.BlockSpec((BR, D), _rows), pl.BlockSpec((BR, 1), _rows),
                  vec, vec, vec, vec, vec],
        out_specs=pl.BlockSpec((BR, D), _rows),
        out_shape=jax.ShapeDtypeStruct((N, D), jnp.float32))(
            a0, a1, hp, dis, b, g, bt, m, v)


def _tc_final(a0, a1, hph, dis, w3, b3):
    def body(a0_ref, a1_ref, hp_ref, dis_ref, w_ref, b_ref, o_ref):
        z = (a0_ref[...] + a1_ref[...] + hp_ref[...]) * dis_ref[...]
        o = lax.dot_general(z, w_ref[...],
                            (((1,), (0,)), ((), ())), **MM_KW) + b_ref[...]
        mx = jnp.max(o, axis=1, keepdims=True)
        lse = jnp.log(jnp.sum(jnp.exp(o - mx), axis=1, keepdims=True))
        o_ref[...] = o - mx - lse
    return pl.pallas_call(
        body, grid=(GRID,),
        in_specs=[pl.BlockSpec((BR, D), _rows), pl.BlockSpec((BR, D), _rows),
                  pl.BlockSpec((BR, D), _rows), pl.BlockSpec((BR, 1), _rows),
                  pl.BlockSpec((D, OUT), _full), pl.BlockSpec((1, OUT), _full)],
        out_specs=pl.BlockSpec((BR, OUT), _rows),
        out_shape=jax.ShapeDtypeStruct((N, OUT), jnp.float32))(
            a0, a1, hph, dis, w3, b3)


def kernel(x, edge_index, W1, b1, W2, b2, W3, b3,
           g1, bt1, m1, v1, g2, bt2, m2, v2):
    src = edge_index[0]
    dst = edge_index[1]

    degp = _sc_degree(dst)
    u1 = _tc_mm(x, W1)
    deg3 = degp.reshape(NC, N, 1)
    dis, hp1 = _tc_scale(deg3, u1)

    a0, a1 = _sc_agg(hp1, src, dst)
    hp2 = _tc_layer(a0, a1, hp1, dis, b1.reshape(1, D),
                    g1.reshape(1, D), bt1.reshape(1, D),
                    m1.reshape(1, D), v1.reshape(1, D), W2)

    a0, a1 = _sc_agg(hp2, src, dst)
    hph2 = _tc_layer_now(a0, a1, hp2, dis, b2.reshape(1, D),
                         g2.reshape(1, D), bt2.reshape(1, D),
                         m2.reshape(1, D), v2.reshape(1, D))

    a0, a1 = _sc_agg(hph2, src, dst)
    return _tc_final(a0, a1, hph2, dis, W3, b3.reshape(1, OUT))


# speedup vs baseline: 1.0825x; 1.0373x over previous
import functools

import jax
import jax.numpy as jnp
from jax import lax
from jax.experimental import pallas as pl
from jax.experimental.pallas import tpu as pltpu
from jax.experimental.pallas import tpu_sc as plsc

N = 10000
E = 320000
D = 128
OUT = 40
EPS = 1e-5

NC = 2
NS = 16
E_PER_TILE = E // (NC * NS)
CHUNK = 128
NCHUNKS = E_PER_TILE // CHUNK
TAIL = E_PER_TILE - NCHUNKS * CHUNK
ZROWS = 40
WB_ROWS = 1000

_mesh = plsc.VectorSubcoreMesh(core_axis_name="c", subcore_axis_name="s")

MM_KW = dict(preferred_element_type=jnp.float32,
             precision=jax.lax.Precision.HIGHEST)


DCHUNK = 80
DNCHUNKS = E_PER_TILE // DCHUNK


@functools.partial(
    pl.kernel, mesh=_mesh,
    out_type=jax.ShapeDtypeStruct((NC * N,), jnp.float32),
    scratch_types=[
        pltpu.VMEM((E_PER_TILE,), jnp.int32),
        pltpu.VMEM((DCHUNK,), jnp.int32),
        pltpu.VMEM((DCHUNK,), jnp.float32),
        pltpu.VMEM((1024,), jnp.float32),
        pltpu.VMEM_SHARED((10240,), jnp.float32),
        pltpu.SemaphoreType.DMA,
    ])
def _sc_degree(dst_hbm, out_hbm, dst_v, dchunk, ones_v, zbuf, acc, sem):
    c = lax.axis_index("c")
    s = lax.axis_index("s")
    t = c * NS + s
    pltpu.sync_copy(dst_hbm.at[pl.ds(t * E_PER_TILE, E_PER_TILE)], dst_v)
    z16 = jnp.zeros((16,), jnp.float32)
    o16 = jnp.ones((16,), jnp.float32)

    @pl.loop(0, DCHUNK, step=16)
    def _(i):
        ones_v[pl.ds(i, 16)] = o16

    @pl.loop(0, 1024, step=16)
    def _(i):
        zbuf[pl.ds(i, 16)] = z16

    @pl.when(s < 10)
    def _():
        pltpu.sync_copy(zbuf, acc.at[pl.ds(s * 1024, 1024)])

    plsc.subcore_barrier()

    @pl.loop(0, DNCHUNKS)
    def _(k):
        @pl.loop(0, DCHUNK, step=16)
        def _(i):
            dchunk[pl.ds(i, 16)] = dst_v[pl.ds(k * DCHUNK + i, 16)]
        pltpu.sync_copy(ones_v, acc.at[dchunk], add=True)

    plsc.subcore_barrier()

    @pl.when(s < 10)
    def _():
        pltpu.sync_copy(acc.at[pl.ds(s * WB_ROWS, WB_ROWS)],
                        zbuf.at[pl.ds(0, WB_ROWS)])
        pltpu.sync_copy(zbuf.at[pl.ds(0, WB_ROWS)],
                        out_hbm.at[pl.ds(c * N + s * WB_ROWS, WB_ROWS)])


NB = 3


@functools.partial(
    pl.kernel, mesh=_mesh,
    out_type=(jax.ShapeDtypeStruct((N, D), jnp.float32),
              jax.ShapeDtypeStruct((N, D), jnp.float32)),
    scratch_types=[
        pltpu.VMEM((NB, CHUNK), jnp.int32),
        pltpu.VMEM((NB, CHUNK), jnp.int32),
        pltpu.VMEM((TAIL,), jnp.int32),
        pltpu.VMEM((TAIL,), jnp.int32),
        pltpu.VMEM((NB, CHUNK, D), jnp.float32),
        pltpu.VMEM_SHARED((N, D), jnp.float32),
        pltpu.SemaphoreType.DMA,
        pltpu.SemaphoreType.DMA,
        pltpu.SemaphoreType.DMA,
        pltpu.SemaphoreType.DMA,
        pltpu.SemaphoreType.DMA,
        pltpu.SemaphoreType.DMA,
        pltpu.SemaphoreType.DMA,
        pltpu.SemaphoreType.DMA,
        pltpu.SemaphoreType.DMA,
        pltpu.SemaphoreType.DMA,
        pltpu.SemaphoreType.DMA,
        pltpu.SemaphoreType.DMA,
    ])
def _sc_agg(hp_hbm, src_hbm, dst_hbm, out0_hbm, out1_hbm,
            schunk, dchunk, stail, dtail, rows_v, acc,
            i0, i1, i2, d0, d1, d2, g0, g1, g2, s0, s1, s2):
    c = lax.axis_index("c")
    s = lax.axis_index("s")
    t = c * NS + s
    base = t * E_PER_TILE
    isem = (i0, i1, i2)
    dsem = (d0, d1, d2)
    gsem = (g0, g1, g2)
    ssem = (s0, s1, s2)

    z16 = jnp.zeros((16,), jnp.float32)

    @pl.loop(0, CHUNK)
    def _(r):
        @pl.loop(0, D, step=16)
        def _(cc):
            rows_v[0, r, pl.ds(cc, 16)] = z16

    @pl.when(s < 10)
    def _():
        @pl.loop(0, 7)
        def _(i):
            pltpu.sync_copy(rows_v.at[0],
                            acc.at[pl.ds(s * WB_ROWS + i * CHUNK, CHUNK)])
        pltpu.sync_copy(rows_v.at[0, pl.ds(0, 104)],
                        acc.at[pl.ds(s * WB_ROWS + 7 * CHUNK, 104)])

    plsc.subcore_barrier()

    pend = {}
    for k in range(NCHUNKS + 2):
        b = k % NB
        if k < NCHUNKS:
            if k >= NB:
                pend[("s", b)].wait()
            pend[("i", b)] = pltpu.async_copy(
                src_hbm.at[pl.ds(base + k * CHUNK, CHUNK)], schunk.at[b],
                isem[b])
            pend[("d", b)] = pltpu.async_copy(
                dst_hbm.at[pl.ds(base + k * CHUNK, CHUNK)], dchunk.at[b],
                dsem[b])
        j = k - 1
        if 0 <= j < NCHUNKS:
            bj = j % NB
            pend[("i", bj)].wait()
            pend[("g", bj)] = pltpu.async_copy(
                hp_hbm.at[schunk.at[bj]], rows_v.at[bj], gsem[bj])
        j2 = k - 2
        if 0 <= j2 < NCHUNKS:
            b2 = j2 % NB
            pend[("g", b2)].wait()
            pend[("d", b2)].wait()
            pend[("s", b2)] = pltpu.async_copy(
                rows_v.at[b2], acc.at[dchunk.at[b2]], ssem[b2], add=True)
    for j in range(max(0, NCHUNKS - NB), NCHUNKS):
        pend[("s", j % NB)].wait()

    pltpu.sync_copy(src_hbm.at[pl.ds(base + NCHUNKS * CHUNK, TAIL)], stail)
    pltpu.sync_copy(dst_hbm.at[pl.ds(base + NCHUNKS * CHUNK, TAIL)], dtail)
    pltpu.sync_copy(hp_hbm.at[stail], rows_v.at[0, pl.ds(0, TAIL)])
    pltpu.sync_copy(rows_v.at[0, pl.ds(0, TAIL)], acc.at[dtail], add=True)

    plsc.subcore_barrier()

    @pl.when(s < 10)
    def _():
        @pl.loop(0, 7)
        def _(i):
            pltpu.sync_copy(acc.at[pl.ds(s * WB_ROWS + i * CHUNK, CHUNK)],
                            rows_v.at[0])

            @pl.when(c == 0)
            def _():
                pltpu.sync_copy(
                    rows_v.at[0],
                    out0_hbm.at[pl.ds(s * WB_ROWS + i * CHUNK, CHUNK)])

            @pl.when(c == 1)
            def _():
                pltpu.sync_copy(
                    rows_v.at[0],
                    out1_hbm.at[pl.ds(s * WB_ROWS + i * CHUNK, CHUNK)])
        pltpu.sync_copy(acc.at[pl.ds(s * WB_ROWS + 7 * CHUNK, 104)],
                        rows_v.at[0, pl.ds(0, 104)])

        @pl.when(c == 0)
        def _():
            pltpu.sync_copy(rows_v.at[0, pl.ds(0, 104)],
                            out0_hbm.at[pl.ds(s * WB_ROWS + 7 * CHUNK, 104)])

        @pl.when(c == 1)
        def _():
            pltpu.sync_copy(rows_v.at[0, pl.ds(0, 104)],
                            out1_hbm.at[pl.ds(s * WB_ROWS + 7 * CHUNK, 104)])


BR = 2000
GRID = N // BR

def _rows(i):
    return (i, 0)

def _full(i):
    return (0, 0)


def _tc_edges(ei):
    def body(e_ref, s_ref, d_ref):
        s_ref[...] = e_ref[0]
        d_ref[...] = e_ref[1]
    return pl.pallas_call(
        body,
        out_shape=(jax.ShapeDtypeStruct((E,), jnp.int32),
                   jax.ShapeDtypeStruct((E,), jnp.int32)))(ei)


def _tc_mm(x, w):
    def body(x_ref, w_ref, o_ref):
        o_ref[...] = lax.dot_general(x_ref[...], w_ref[...],
                                     (((1,), (0,)), ((), ())), **MM_KW)
    return pl.pallas_call(
        body, grid=(GRID,),
        in_specs=[pl.BlockSpec((BR, D), _rows), pl.BlockSpec((D, D), _full)],
        out_specs=pl.BlockSpec((BR, D), _rows),
        out_shape=jax.ShapeDtypeStruct((N, D), jnp.float32))(x, w)


def _tc_scale(deg3, u1):
    def body(deg_ref, u_ref, dis_ref, hp_ref):
        dis = lax.rsqrt(deg_ref[0] + deg_ref[1] + 1.0)
        dis_ref[...] = dis
        hp_ref[...] = dis * u_ref[...]
    return pl.pallas_call(
        body, grid=(GRID,),
        in_specs=[pl.BlockSpec((2, BR, 1), lambda i: (0, i, 0)),
                  pl.BlockSpec((BR, D), _rows)],
        out_specs=(pl.BlockSpec((BR, 1), _rows), pl.BlockSpec((BR, D), _rows)),
        out_shape=(jax.ShapeDtypeStruct((N, 1), jnp.float32),
                   jax.ShapeDtypeStruct((N, D), jnp.float32)))(deg3, u1)


def _tc_layer(a0, a1, hp, dis, b, g, bt, m, v, w_next):
    def body(a0_ref, a1_ref, hp_ref, dis_ref, b_ref, g_ref, bt_ref, m_ref,
             v_ref, w_ref, o_ref):
        dis = dis_ref[...]
        z = (a0_ref[...] + a1_ref[...] + hp_ref[...]) * dis + b_ref[...]
        sc = g_ref[...] * lax.rsqrt(v_ref[...] + EPS)
        sh = bt_ref[...] - m_ref[...] * sc
        h = jnp.maximum(z * sc + sh, 0.0)
        o_ref[...] = dis * lax.dot_general(h, w_ref[...],
                                           (((1,), (0,)), ((), ())), **MM_KW)
    vec = pl.BlockSpec((1, D), _full)
    return pl.pallas_call(
        body, grid=(GRID,),
        in_specs=[pl.BlockSpec((BR, D), _rows), pl.BlockSpec((BR, D), _rows),
                  pl.BlockSpec((BR, D), _rows), pl.BlockSpec((BR, 1), _rows),
                  vec, vec, vec, vec, vec, pl.BlockSpec((D, D), _full)],
        out_specs=pl.BlockSpec((BR, D), _rows),
        out_shape=jax.ShapeDtypeStruct((N, D), jnp.float32))(
            a0, a1, hp, dis, b, g, bt, m, v, w_next)


def _tc_layer_now(a0, a1, hp, dis, b, g, bt, m, v):
    def body(a0_ref, a1_ref, hp_ref, dis_ref, b_ref, g_ref, bt_ref, m_ref,
             v_ref, o_ref):
        dis = dis_ref[...]
        z = (a0_ref[...] + a1_ref[...] + hp_ref[...]) * dis + b_ref[...]
        sc = g_ref[...] * lax.rsqrt(v_ref[...] + EPS)
        sh = bt_ref[...] - m_ref[...] * sc
        o_ref[...] = dis * jnp.maximum(z * sc + sh, 0.0)
    vec = pl.BlockSpec((1, D), _full)
    return pl.pallas_call(
        body, grid=(GRID,),
        in_specs=[pl.BlockSpec((BR, D), _rows), pl.BlockSpec((BR, D), _rows),
                  pl.BlockSpec((BR, D), _rows), pl.BlockSpec((BR, 1), _rows),
                  vec, vec, vec, vec, vec],
        out_specs=pl.BlockSpec((BR, D), _rows),
        out_shape=jax.ShapeDtypeStruct((N, D), jnp.float32))(
            a0, a1, hp, dis, b, g, bt, m, v)


def _tc_final(a0, a1, hph, dis, w3, b3):
    def body(a0_ref, a1_ref, hp_ref, dis_ref, w_ref, b_ref, o_ref):
        z = (a0_ref[...] + a1_ref[...] + hp_ref[...]) * dis_ref[...]
        o = lax.dot_general(z, w_ref[...],
                            (((1,), (0,)), ((), ())), **MM_KW) + b_ref[...]
        mx = jnp.max(o, axis=1, keepdims=True)
        lse = jnp.log(jnp.sum(jnp.exp(o - mx), axis=1, keepdims=True))
        o_ref[...] = (o - mx - lse).T
    return pl.pallas_call(
        body,
        out_shape=jax.ShapeDtypeStruct((OUT, N), jnp.float32))(
            a0, a1, hph, dis, w3, b3)


def kernel(x, edge_index, W1, b1, W2, b2, W3, b3,
           g1, bt1, m1, v1, g2, bt2, m2, v2):
    src, dst = _tc_edges(edge_index)

    degp = _sc_degree(dst)
    u1 = _tc_mm(x, W1)
    deg3 = degp.reshape(NC, N, 1)
    dis, hp1 = _tc_scale(deg3, u1)

    a0, a1 = _sc_agg(hp1, src, dst)
    hp2 = _tc_layer(a0, a1, hp1, dis, b1.reshape(1, D),
                    g1.reshape(1, D), bt1.reshape(1, D),
                    m1.reshape(1, D), v1.reshape(1, D), W2)

    a0, a1 = _sc_agg(hp2, src, dst)
    hph2 = _tc_layer_now(a0, a1, hp2, dis, b2.reshape(1, D),
                         g2.reshape(1, D), bt2.reshape(1, D),
                         m2.reshape(1, D), v2.reshape(1, D))

    a0, a1 = _sc_agg(hph2, src, dst)
    return _tc_final(a0, a1, hph2, dis, W3, b3.reshape(1, OUT)).T


# speedup vs baseline: 1.1184x; 1.0332x over previous
import functools

import jax
import jax.numpy as jnp
from jax import lax
from jax.experimental import pallas as pl
from jax.experimental.pallas import tpu as pltpu
from jax.experimental.pallas import tpu_sc as plsc

N = 10000
E = 320000
D = 128
OUT = 40
EPS = 1e-5

NC = 2
NS = 16
E_PER_TILE = E // (NC * NS)
CHUNK = 128
NCHUNKS = E_PER_TILE // CHUNK
TAIL = E_PER_TILE - NCHUNKS * CHUNK
ZROWS = 40
WB_ROWS = 1000

_mesh = plsc.VectorSubcoreMesh(core_axis_name="c", subcore_axis_name="s")

MM_KW = dict(preferred_element_type=jnp.float32,
             precision=jax.lax.Precision.HIGHEST)


DCHUNK = 80
DNCHUNKS = E_PER_TILE // DCHUNK


@functools.partial(
    pl.kernel, mesh=_mesh,
    out_type=jax.ShapeDtypeStruct((2 * 10240,), jnp.float32),
    scratch_types=[
        pltpu.VMEM((E_PER_TILE,), jnp.int32),
        pltpu.VMEM((DCHUNK,), jnp.int32),
        pltpu.VMEM((DCHUNK,), jnp.float32),
        pltpu.VMEM((1024,), jnp.float32),
        pltpu.VMEM_SHARED((10240,), jnp.float32),
        pltpu.SemaphoreType.DMA,
    ])
def _sc_degree(dst_hbm, out_hbm, dst_v, dchunk, ones_v, zbuf, acc, sem):
    c = lax.axis_index("c")
    s = lax.axis_index("s")
    t = c * NS + s
    pltpu.sync_copy(dst_hbm.at[pl.ds(t * E_PER_TILE, E_PER_TILE)], dst_v)
    z16 = jnp.zeros((16,), jnp.float32)
    o16 = jnp.ones((16,), jnp.float32)

    @pl.loop(0, DCHUNK, step=16)
    def _(i):
        ones_v[pl.ds(i, 16)] = o16

    @pl.loop(0, 1024, step=16)
    def _(i):
        zbuf[pl.ds(i, 16)] = z16

    @pl.when(s < 10)
    def _():
        pltpu.sync_copy(zbuf, acc.at[pl.ds(s * 1024, 1024)])

    plsc.subcore_barrier()

    @pl.loop(0, DNCHUNKS)
    def _(k):
        @pl.loop(0, DCHUNK, step=16)
        def _(i):
            dchunk[pl.ds(i, 16)] = dst_v[pl.ds(k * DCHUNK + i, 16)]
        pltpu.sync_copy(ones_v, acc.at[dchunk], add=True)

    plsc.subcore_barrier()

    @pl.when(s < 10)
    def _():
        pltpu.sync_copy(acc.at[pl.ds(s * WB_ROWS, WB_ROWS)],
                        zbuf.at[pl.ds(0, WB_ROWS)])
        pltpu.sync_copy(zbuf.at[pl.ds(0, WB_ROWS)],
                        out_hbm.at[pl.ds(c * 10240 + s * WB_ROWS, WB_ROWS)])


NB = 3


@functools.partial(
    pl.kernel, mesh=_mesh,
    out_type=(jax.ShapeDtypeStruct((N, D), jnp.float32),
              jax.ShapeDtypeStruct((N, D), jnp.float32)),
    scratch_types=[
        pltpu.VMEM((NB, CHUNK), jnp.int32),
        pltpu.VMEM((NB, CHUNK), jnp.int32),
        pltpu.VMEM((TAIL,), jnp.int32),
        pltpu.VMEM((TAIL,), jnp.int32),
        pltpu.VMEM((NB, CHUNK, D), jnp.float32),
        pltpu.VMEM_SHARED((N, D), jnp.float32),
        pltpu.SemaphoreType.DMA,
        pltpu.SemaphoreType.DMA,
        pltpu.SemaphoreType.DMA,
        pltpu.SemaphoreType.DMA,
        pltpu.SemaphoreType.DMA,
        pltpu.SemaphoreType.DMA,
        pltpu.SemaphoreType.DMA,
        pltpu.SemaphoreType.DMA,
        pltpu.SemaphoreType.DMA,
        pltpu.SemaphoreType.DMA,
        pltpu.SemaphoreType.DMA,
        pltpu.SemaphoreType.DMA,
    ])
def _sc_agg(hp_hbm, src_hbm, dst_hbm, out0_hbm, out1_hbm,
            schunk, dchunk, stail, dtail, rows_v, acc,
            i0, i1, i2, d0, d1, d2, g0, g1, g2, s0, s1, s2):
    c = lax.axis_index("c")
    s = lax.axis_index("s")
    t = c * NS + s
    base = t * E_PER_TILE
    isem = (i0, i1, i2)
    dsem = (d0, d1, d2)
    gsem = (g0, g1, g2)
    ssem = (s0, s1, s2)

    z16 = jnp.zeros((16,), jnp.float32)

    @pl.loop(0, CHUNK)
    def _(r):
        @pl.loop(0, D, step=16)
        def _(cc):
            rows_v[0, r, pl.ds(cc, 16)] = z16

    @pl.when(s < 10)
    def _():
        @pl.loop(0, 7)
        def _(i):
            pltpu.sync_copy(rows_v.at[0],
                            acc.at[pl.ds(s * WB_ROWS + i * CHUNK, CHUNK)])
        pltpu.sync_copy(rows_v.at[0, pl.ds(0, 104)],
                        acc.at[pl.ds(s * WB_ROWS + 7 * CHUNK, 104)])

    plsc.subcore_barrier()

    pend = {}
    for k in range(NCHUNKS + 2):
        b = k % NB
        if k < NCHUNKS:
            if k >= NB:
                pend[("s", b)].wait()
            pend[("i", b)] = pltpu.async_copy(
                src_hbm.at[pl.ds(base + k * CHUNK, CHUNK)], schunk.at[b],
                isem[b])
            pend[("d", b)] = pltpu.async_copy(
                dst_hbm.at[pl.ds(base + k * CHUNK, CHUNK)], dchunk.at[b],
                dsem[b])
        j = k - 1
        if 0 <= j < NCHUNKS:
            bj = j % NB
            pend[("i", bj)].wait()
            pend[("g", bj)] = pltpu.async_copy(
                hp_hbm.at[schunk.at[bj]], rows_v.at[bj], gsem[bj])
        j2 = k - 2
        if 0 <= j2 < NCHUNKS:
            b2 = j2 % NB
            pend[("g", b2)].wait()
            pend[("d", b2)].wait()
            pend[("s", b2)] = pltpu.async_copy(
                rows_v.at[b2], acc.at[dchunk.at[b2]], ssem[b2], add=True)
    for j in range(max(0, NCHUNKS - NB), NCHUNKS):
        pend[("s", j % NB)].wait()

    pltpu.sync_copy(src_hbm.at[pl.ds(base + NCHUNKS * CHUNK, TAIL)], stail)
    pltpu.sync_copy(dst_hbm.at[pl.ds(base + NCHUNKS * CHUNK, TAIL)], dtail)
    pltpu.sync_copy(hp_hbm.at[stail], rows_v.at[0, pl.ds(0, TAIL)])
    pltpu.sync_copy(rows_v.at[0, pl.ds(0, TAIL)], acc.at[dtail], add=True)

    plsc.subcore_barrier()

    @pl.when(s < 10)
    def _():
        @pl.loop(0, 7)
        def _(i):
            pltpu.sync_copy(acc.at[pl.ds(s * WB_ROWS + i * CHUNK, CHUNK)],
                            rows_v.at[0])

            @pl.when(c == 0)
            def _():
                pltpu.sync_copy(
                    rows_v.at[0],
                    out0_hbm.at[pl.ds(s * WB_ROWS + i * CHUNK, CHUNK)])

            @pl.when(c == 1)
            def _():
                pltpu.sync_copy(
                    rows_v.at[0],
                    out1_hbm.at[pl.ds(s * WB_ROWS + i * CHUNK, CHUNK)])
        pltpu.sync_copy(acc.at[pl.ds(s * WB_ROWS + 7 * CHUNK, 104)],
                        rows_v.at[0, pl.ds(0, 104)])

        @pl.when(c == 0)
        def _():
            pltpu.sync_copy(rows_v.at[0, pl.ds(0, 104)],
                            out0_hbm.at[pl.ds(s * WB_ROWS + 7 * CHUNK, 104)])

        @pl.when(c == 1)
        def _():
            pltpu.sync_copy(rows_v.at[0, pl.ds(0, 104)],
                            out1_hbm.at[pl.ds(s * WB_ROWS + 7 * CHUNK, 104)])


BR = 2000
GRID = N // BR

def _rows(i):
    return (i, 0)

def _full(i):
    return (0, 0)


def _tc_edges(ei):
    def body(e_ref, s_ref, d_ref):
        s_ref[...] = e_ref[0]
        d_ref[...] = e_ref[1]
    return pl.pallas_call(
        body,
        out_shape=(jax.ShapeDtypeStruct((E,), jnp.int32),
                   jax.ShapeDtypeStruct((E,), jnp.int32)))(ei)


def _tc_mm(x, w):
    def body(x_ref, w_ref, o_ref):
        o_ref[...] = lax.dot_general(x_ref[...], w_ref[...],
                                     (((1,), (0,)), ((), ())), **MM_KW)
    return pl.pallas_call(
        body, grid=(GRID,),
        in_specs=[pl.BlockSpec((BR, D), _rows), pl.BlockSpec((D, D), _full)],
        out_specs=pl.BlockSpec((BR, D), _rows),
        out_shape=jax.ShapeDtypeStruct((N, D), jnp.float32))(x, w)


def _tc_scale(degp, u1):
    def body(dp_ref, u_ref, dis_ref, hp_ref):
        d = dp_ref[pl.ds(0, 10240)] + dp_ref[pl.ds(10240, 10240)] + 1.0
        dis = lax.rsqrt(d)[0:N].reshape(N, 1)
        dis_ref[...] = dis
        hp_ref[...] = dis * u_ref[...]
    return pl.pallas_call(
        body,
        out_shape=(jax.ShapeDtypeStruct((N, 1), jnp.float32),
                   jax.ShapeDtypeStruct((N, D), jnp.float32)))(degp, u1)


def _tc_layer(a0, a1, hp, dis, b, g, bt, m, v, w_next):
    def body(a0_ref, a1_ref, hp_ref, dis_ref, b_ref, g_ref, bt_ref, m_ref,
             v_ref, w_ref, o_ref):
        dis = dis_ref[...]
        z = (a0_ref[...] + a1_ref[...] + hp_ref[...]) * dis + b_ref[...]
        sc = g_ref[...] * lax.rsqrt(v_ref[...] + EPS)
        sh = bt_ref[...] - m_ref[...] * sc
        h = jnp.maximum(z * sc + sh, 0.0)
        o_ref[...] = dis * lax.dot_general(h, w_ref[...],
                                           (((1,), (0,)), ((), ())), **MM_KW)
    vec = pl.BlockSpec((1, D), _full)
    return pl.pallas_call(
        body, grid=(GRID,),
        in_specs=[pl.BlockSpec((BR, D), _rows), pl.BlockSpec((BR, D), _rows),
                  pl.BlockSpec((BR, D), _rows), pl.BlockSpec((BR, 1), _rows),
                  vec, vec, vec, vec, vec, pl.BlockSpec((D, D), _full)],
        out_specs=pl.BlockSpec((BR, D), _rows),
        out_shape=jax.ShapeDtypeStruct((N, D), jnp.float32))(
            a0, a1, hp, dis, b, g, bt, m, v, w_next)


def _tc_layer_now(a0, a1, hp, dis, b, g, bt, m, v):
    def body(a0_ref, a1_ref, hp_ref, dis_ref, b_ref, g_ref, bt_ref, m_ref,
             v_ref, o_ref):
        dis = dis_ref[...]
        z = (a0_ref[...] + a1_ref[...] + hp_ref[...]) * dis + b_ref[...]
        sc = g_ref[...] * lax.rsqrt(v_ref[...] + EPS)
        sh = bt_ref[...] - m_ref[...] * sc
        o_ref[...] = dis * jnp.maximum(z * sc + sh, 0.0)
    vec = pl.BlockSpec((1, D), _full)
    return pl.pallas_call(
        body, grid=(GRID,),
        in_specs=[pl.BlockSpec((BR, D), _rows), pl.BlockSpec((BR, D), _rows),
                  pl.BlockSpec((BR, D), _rows), pl.BlockSpec((BR, 1), _rows),
                  vec, vec, vec, vec, vec],
        out_specs=pl.BlockSpec((BR, D), _rows),
        out_shape=jax.ShapeDtypeStruct((N, D), jnp.float32))(
            a0, a1, hp, dis, b, g, bt, m, v)


def _tc_final(a0, a1, hph, dis, w3, b3):
    def body(a0_ref, a1_ref, hp_ref, dis_ref, w_ref, b_ref, o_ref):
        z = (a0_ref[...] + a1_ref[...] + hp_ref[...]) * dis_ref[...]
        o = lax.dot_general(z, w_ref[...],
                            (((1,), (0,)), ((), ())), **MM_KW) + b_ref[...]
        mx = jnp.max(o, axis=1, keepdims=True)
        lse = jnp.log(jnp.sum(jnp.exp(o - mx), axis=1, keepdims=True))
        o_ref[...] = (o - mx - lse).T
    return pl.pallas_call(
        body,
        out_shape=jax.ShapeDtypeStruct((OUT, N), jnp.float32))(
            a0, a1, hph, dis, w3, b3)


def kernel(x, edge_index, W1, b1, W2, b2, W3, b3,
           g1, bt1, m1, v1, g2, bt2, m2, v2):
    src, dst = _tc_edges(edge_index)

    degp = _sc_degree(dst)
    u1 = _tc_mm(x, W1)
    dis, hp1 = _tc_scale(degp, u1)

    a0, a1 = _sc_agg(hp1, src, dst)
    hp2 = _tc_layer(a0, a1, hp1, dis, b1.reshape(1, D),
                    g1.reshape(1, D), bt1.reshape(1, D),
                    m1.reshape(1, D), v1.reshape(1, D), W2)

    a0, a1 = _sc_agg(hp2, src, dst)
    hph2 = _tc_layer_now(a0, a1, hp2, dis, b2.reshape(1, D),
                         g2.reshape(1, D), bt2.reshape(1, D),
                         m2.reshape(1, D), v2.reshape(1, D))

    a0, a1 = _sc_agg(hph2, src, dst)
    return _tc_final(a0, a1, hph2, dis, W3, b3.reshape(1, OUT)).T
